# Initial kernel scaffold; baseline (speedup 1.0000x reference)
import jax
import jax.numpy as jnp
from jax.experimental import pallas as pl


def kernel(input_features, edge_index0, edge_index1, W_self0, W_neigh0, b0, W_self1, W_neigh1, b1):
    raise NotImplementedError("write your pallas kernel here")


# speedup vs baseline: 8.0561x; 8.0561x over previous
import functools

import jax
import jax.numpy as jnp
from jax import lax
from jax.experimental import pallas as pl
from jax.experimental.pallas import tpu as pltpu
from jax.experimental.pallas import tpu_sc as plsc

N = 10000
D = 128
E = 320000
NC = 2
NS = 16
NW = NC * NS
EPW = E // NW
K = 80
CHUNKS = EPW // K
NP = 10240
RPT = NP // NS
ZR = 64


def _sc_agg_body(h_hbm, src_hbm, dst_hbm, acc_out, deg_out,
                 idxa_v, idxb_v, rows_v, ones_v, zbuf_v, zdeg_v,
                 acc_sh, deg_sh, sem0, sem1):
    idxbufs = (idxa_v, idxb_v)
    gsems = (sem0, sem1)
    c = lax.axis_index("c")
    s = lax.axis_index("s")
    wid = c * NS + s

    zv = jnp.zeros((16,), jnp.float32)
    ov = jnp.ones((16,), jnp.float32)

    @pl.loop(0, ZR)
    def _zero_bufs(i):
        for j in range(D // 16):
            zbuf_v[i, pl.ds(j * 16, 16)] = zv

    @pl.loop(0, RPT // 16)
    def _zero_deg(i):
        zdeg_v[pl.ds(i * 16, 16)] = zv

    @pl.loop(0, K // 16)
    def _init_ones(i):
        ones_v[pl.ds(i * 16, 16)] = ov

    row0 = s * RPT
    for t in range(RPT // ZR):
        pltpu.sync_copy(zbuf_v, acc_sh.at[pl.ds(row0 + t * ZR, ZR)])
    pltpu.sync_copy(zdeg_v, deg_sh.at[pl.ds(row0, RPT)])
    plsc.subcore_barrier()

    base = wid * EPW

    def load_idx(j, buf):
        off = base + j * K
        pltpu.sync_copy(src_hbm.at[pl.ds(off, K)], buf.at[0])
        pltpu.sync_copy(dst_hbm.at[pl.ds(off, K)], buf.at[1])

    def fire_gather(buf, slot):
        pltpu.async_copy(h_hbm.at[buf.at[0]], rows_v.at[slot], gsems[slot])

    def consume(buf, slot):
        pltpu.make_async_copy(h_hbm.at[buf.at[0]], rows_v.at[slot],
                              gsems[slot]).wait()
        pltpu.sync_copy(rows_v.at[slot], acc_sh.at[buf.at[1]], add=True)
        pltpu.sync_copy(ones_v, deg_sh.at[buf.at[1]], add=True)

    load_idx(0, idxa_v)
    fire_gather(idxa_v, 0)

    @pl.loop(0, (CHUNKS - 1) // 2)
    def _edges(i):
        for par in range(2):
            j = i * 2 + par
            load_idx(j + 1, idxbufs[1 - par])
            fire_gather(idxbufs[1 - par], 1 - par)
            consume(idxbufs[par], par)

    consume(idxbufs[(CHUNKS - 1) % 2], (CHUNKS - 1) % 2)

    plsc.subcore_barrier()
    pltpu.sync_copy(acc_sh.at[pl.ds(row0, RPT)], acc_out.at[c, pl.ds(row0, RPT)])
    pltpu.sync_copy(deg_sh.at[pl.ds(row0, RPT)], deg_out.at[c, pl.ds(row0, RPT)])


@functools.lru_cache(maxsize=None)
def _make_sc_agg():
    return pl.kernel(
        _sc_agg_body,
        out_type=(
            jax.ShapeDtypeStruct((NC, NP, D), jnp.float32),
            jax.ShapeDtypeStruct((NC, NP), jnp.float32),
        ),
        mesh=plsc.VectorSubcoreMesh(core_axis_name="c", subcore_axis_name="s",
                                    num_cores=NC, num_subcores=NS),
        scratch_types=[
            pltpu.VMEM((2, K), jnp.int32),
            pltpu.VMEM((2, K), jnp.int32),
            pltpu.VMEM((2, K, D), jnp.float32),
            pltpu.VMEM((K,), jnp.float32),
            pltpu.VMEM((ZR, D), jnp.float32),
            pltpu.VMEM((RPT,), jnp.float32),
            pltpu.VMEM_SHARED((NP, D), jnp.float32),
            pltpu.VMEM_SHARED((NP,), jnp.float32),
            pltpu.SemaphoreType.DMA,
            pltpu.SemaphoreType.DMA,
        ],
    )


def _mm_body(relu, x_ref, a0_ref, a1_ref, d0_ref, d1_ref,
             ws_ref, wn_ref, b_ref, o_ref):
    x = x_ref[...]
    a = a0_ref[...] + a1_ref[...]
    deg = jnp.clip(d0_ref[...] + d1_ref[...], 1.0, None)
    mean = a / deg
    out = (jnp.dot(x, ws_ref[...], preferred_element_type=jnp.float32)
           + jnp.dot(mean, wn_ref[...], preferred_element_type=jnp.float32)
           + b_ref[...])
    if relu:
        out = jnp.maximum(out, 0.0)
    o_ref[...] = out


def _mm(relu, x, a0, a1, d0, d1, ws, wn, b):
    R = 1000
    grid = (N // R,)
    return pl.pallas_call(
        functools.partial(_mm_body, relu),
        grid=grid,
        in_specs=[
            pl.BlockSpec((R, D), lambda i: (i, 0)),
            pl.BlockSpec((R, D), lambda i: (i, 0)),
            pl.BlockSpec((R, D), lambda i: (i, 0)),
            pl.BlockSpec((R, 1), lambda i: (i, 0)),
            pl.BlockSpec((R, 1), lambda i: (i, 0)),
            pl.BlockSpec((D, D), lambda i: (0, 0)),
            pl.BlockSpec((D, D), lambda i: (0, 0)),
            pl.BlockSpec((1, D), lambda i: (0, 0)),
        ],
        out_specs=pl.BlockSpec((R, D), lambda i: (i, 0)),
        out_shape=jax.ShapeDtypeStruct((N, D), jnp.float32),
    )(x, a0, a1, d0, d1, ws, wn, b)


def kernel(input_features, edge_index0, edge_index1,
           W_self0, W_neigh0, b0, W_self1, W_neigh1, b1):
    src0 = edge_index0[0].astype(jnp.int32)
    dst0 = edge_index0[1].astype(jnp.int32)
    src1 = edge_index1[0].astype(jnp.int32)
    dst1 = edge_index1[1].astype(jnp.int32)

    sc_agg = _make_sc_agg()
    acc0, deg0 = sc_agg(input_features, src0, dst0)
    h1 = _mm(True, input_features, acc0[0, :N], acc0[1, :N],
             deg0[0, :N].reshape(N, 1), deg0[1, :N].reshape(N, 1),
             W_self0, W_neigh0, b0.reshape(1, D))
    acc1, deg1 = sc_agg(h1, src1, dst1)
    return _mm(False, h1, acc1[0, :N], acc1[1, :N],
               deg1[0, :N].reshape(N, 1), deg1[1, :N].reshape(N, 1),
               W_self1, W_neigh1, b1.reshape(1, D))
